# Initial kernel scaffold; baseline (speedup 1.0000x reference)
import jax
import jax.numpy as jnp
from jax.experimental import pallas as pl


def kernel(atom_embs, edge_indices, pos, edge_weight, Wq, Wk, Wv, Wi, bi, Wj, bj, We, be, Wr, br, Wo, bo, ln_g, ln_b, W1, b1, W2, b2, W3, b3):
    raise NotImplementedError("write your pallas kernel here")


# speedup vs baseline: 2.5166x; 2.5166x over previous
import functools

import jax
import jax.numpy as jnp
import numpy as np
from jax import lax
from jax.experimental import pallas as pl
from jax.experimental.pallas import tpu as pltpu
from jax.experimental.pallas import tpu_sc as plsc

H = 128
NHEAD = 8
HH = H * NHEAD
RBF_K = 64
CUTOFF = 10.0
N_NODES = 1024
N_EDGES = 2048

_NC, _NS = 2, 16
_NW = _NC * _NS
_EPW = N_EDGES // _NW


def _sc_gather(atom_embs, pos_pad, src, dst):
    mesh = plsc.VectorSubcoreMesh(core_axis_name="c", subcore_axis_name="s")

    @functools.partial(
        pl.kernel,
        out_type=(
            jax.ShapeDtypeStruct((N_EDGES, H), jnp.float32),
            jax.ShapeDtypeStruct((N_EDGES, H), jnp.float32),
            jax.ShapeDtypeStruct((N_EDGES, H), jnp.float32),
            jax.ShapeDtypeStruct((N_EDGES, H), jnp.float32),
        ),
        mesh=mesh,
        scratch_types=[
            pltpu.VMEM((_EPW,), jnp.int32),
            pltpu.VMEM((_EPW,), jnp.int32),
            pltpu.VMEM((_EPW, H), jnp.float32),
            pltpu.SemaphoreType.DMA,
        ],
    )
    def k(embs_hbm, pos_hbm, src_hbm, dst_hbm, gd_hbm, gs_hbm, pd_hbm, ps_hbm,
          idx_d, idx_s, rows, sem):
        wid = lax.axis_index("s") * _NC + lax.axis_index("c")
        base = wid * _EPW
        pltpu.sync_copy(dst_hbm.at[pl.ds(base, _EPW)], idx_d)
        pltpu.sync_copy(src_hbm.at[pl.ds(base, _EPW)], idx_s)
        pltpu.async_copy(embs_hbm.at[idx_d], rows, sem).wait()
        pltpu.sync_copy(rows, gd_hbm.at[pl.ds(base, _EPW)])
        pltpu.async_copy(embs_hbm.at[idx_s], rows, sem).wait()
        pltpu.sync_copy(rows, gs_hbm.at[pl.ds(base, _EPW)])
        pltpu.async_copy(pos_hbm.at[idx_d], rows, sem).wait()
        pltpu.sync_copy(rows, pd_hbm.at[pl.ds(base, _EPW)])
        pltpu.async_copy(pos_hbm.at[idx_s], rows, sem).wait()
        pltpu.sync_copy(rows, ps_hbm.at[pl.ds(base, _EPW)])

    return k(atom_embs, pos_pad, src, dst)


def _sc_scatter(msg, dst, zeros):
    mesh = plsc.VectorSubcoreMesh(core_axis_name="c", subcore_axis_name="s")
    rpw = N_NODES // _NS

    @functools.partial(
        pl.kernel,
        out_type=jax.ShapeDtypeStruct((_NC, N_NODES, H), jnp.float32),
        mesh=mesh,
        scratch_types=[
            pltpu.VMEM((_EPW,), jnp.int32),
            pltpu.VMEM((_EPW, H), jnp.float32),
            pltpu.VMEM_SHARED((N_NODES, H), jnp.float32),
            pltpu.SemaphoreType.DMA,
        ],
    )
    def k(msg_hbm, dst_hbm, zeros_hbm, out_hbm, idx_v, rows_v, agg_s, sem):
        cid = lax.axis_index("c")
        sid = lax.axis_index("s")
        wid = sid * _NC + cid
        base = wid * _EPW

        @pl.when(sid == 0)
        def _():
            pltpu.sync_copy(zeros_hbm, agg_s)

        plsc.subcore_barrier()
        pltpu.sync_copy(msg_hbm.at[pl.ds(base, _EPW)], rows_v)
        pltpu.sync_copy(dst_hbm.at[pl.ds(base, _EPW)], idx_v)
        pltpu.sync_copy(rows_v, agg_s.at[idx_v], add=True)
        plsc.subcore_barrier()
        pltpu.sync_copy(agg_s.at[pl.ds(sid * rpw, rpw)],
                        out_hbm.at[cid, pl.ds(sid * rpw, rpw)])

    return k(msg, dst, zeros)


def _dot_t(a, b):
    return lax.dot_general(a, b, (((1,), (1,)), ((), ())),
                           preferred_element_type=jnp.float32)


def _dot(a, b):
    return lax.dot_general(a, b, (((1,), (0,)), ((), ())),
                           preferred_element_type=jnp.float32)


_RBF_WIDTH = float((0.5 / ((1.0 - np.exp(-CUTOFF)) / RBF_K)) ** 2)


def _tc_prep(gd, gs, pd, ps, ew, Wq, Wk, Wv, Wi, bi, Wj, bj, We, be, Wr, br,
             centers, interpret=False):
    blk = 512
    grid = N_EDGES // blk

    def body(gd_r, gs_r, pd_r, ps_r, ew_r, Wq_r, Wk_r, Wv_r, Wi_r, bi_r,
             Wj_r, bj_r, We_r, be_r, Wr_r, br_r, c_r, q_o, k_o, inner_o):
        ew_b = ew_r[...]
        x_i = gd_r[...] + ew_b
        x_j = gs_r[...] + ew_b
        q_o[...] = _dot_t(x_i, Wq_r[...])
        k_o[...] = _dot_t(x_i, Wk_r[...])
        v = _dot_t(x_i, Wv_r[...])
        hi = _dot_t(x_i, Wi_r[...]) + bi_r[...]
        hj = _dot_t(x_j, Wj_r[...]) + bj_r[...]
        edge = jnp.concatenate([hi + hj, hi - hj, hi * hj], axis=1)
        diff = pd_r[...] - ps_r[...]
        dist = jnp.sqrt(jnp.sum(diff * diff, axis=1, keepdims=True))
        x = dist / CUTOFF
        x3 = x ** 3
        x4 = x3 * x
        x5 = x4 * x
        cut = jnp.where(x < 1.0, 1 - 6 * x5 + 15 * x4 - 10 * x3,
                        jnp.zeros_like(x))
        rbf = cut * jnp.exp(-_RBF_WIDTH * (jnp.exp(-dist) - c_r[...]) ** 2)
        inner_o[...] = (_dot_t(edge, We_r[...]) + be_r[...] +
                        _dot_t(rbf, Wr_r[...]) + br_r[...] + v)

    full = lambda shape: pl.BlockSpec(shape, lambda i: (0, 0))
    rows = lambda w: pl.BlockSpec((blk, w), lambda i: (i, 0))
    return pl.pallas_call(
        body,
        grid=(grid,),
        in_specs=[
            rows(H), rows(H), rows(H), rows(H), rows(1),
            full((HH, H)), full((HH, H)), full((HH, H)),
            full((H, H)), full((1, H)), full((H, H)), full((1, H)),
            full((HH, 3 * H)), full((1, HH)), full((HH, RBF_K)),
            full((1, HH)), full((1, RBF_K)),
        ],
        out_specs=[rows(HH), rows(HH), rows(HH)],
        out_shape=[
            jax.ShapeDtypeStruct((N_EDGES, HH), jnp.float32),
            jax.ShapeDtypeStruct((N_EDGES, HH), jnp.float32),
            jax.ShapeDtypeStruct((N_EDGES, HH), jnp.float32),
        ],
        interpret=interpret,
    )(gd, gs, pd, ps, ew, Wq, Wk, Wv, Wi, bi, Wj, bj, We, be, Wr, br, centers)


def _tc_attn(Q, K, inner, src2, Wo, bo, interpret=False):
    blk = 256
    grid = N_EDGES // blk
    scale = float(H) ** -0.5

    def body(q_r, k_r, inner_r, src_r, wo_r, bo_r, msg_o, p_scr):
        @pl.when(pl.program_id(0) == 0)
        def _():
            ids = lax.broadcasted_iota(jnp.int32, (N_EDGES, N_NODES), 1)
            p_scr[...] = (src_r[...] == ids).astype(jnp.float32)

        logits = _dot_t(q_r[...], k_r[...]) * scale
        c = jnp.max(logits, axis=1, keepdims=True)
        e = jnp.exp(logits - c)
        s = _dot(e, p_scr[...])
        denom = _dot_t(s, p_scr[...])
        out = _dot(e / denom, inner_r[...])
        msg_o[...] = _dot_t(out, wo_r[...]) + bo_r[...]

    return pl.pallas_call(
        body,
        grid=(grid,),
        in_specs=[
            pl.BlockSpec((blk, HH), lambda i: (i, 0)),
            pl.BlockSpec((N_EDGES, HH), lambda i: (0, 0)),
            pl.BlockSpec((N_EDGES, HH), lambda i: (0, 0)),
            pl.BlockSpec((N_EDGES, 1), lambda i: (0, 0)),
            pl.BlockSpec((H, HH), lambda i: (0, 0)),
            pl.BlockSpec((1, H), lambda i: (0, 0)),
        ],
        out_specs=pl.BlockSpec((blk, H), lambda i: (i, 0)),
        out_shape=jax.ShapeDtypeStruct((N_EDGES, H), jnp.float32),
        scratch_shapes=[pltpu.VMEM((N_EDGES, N_NODES), jnp.float32)],
        interpret=interpret,
    )(Q, K, inner, src2, Wo, bo)


def _layer_norm_in(x, g, b, eps=1e-5):
    mu = jnp.mean(x, axis=-1, keepdims=True)
    var = jnp.mean((x - mu) ** 2, axis=-1, keepdims=True)
    return (x - mu) / jnp.sqrt(var + eps) * g + b


def _softplus(x):
    return jnp.maximum(x, 0.0) + jnp.log(1.0 + jnp.exp(-jnp.abs(x)))


def _tc_final(aggp, ln_g, ln_b, W1, b1, W2, b2, W3, b3, interpret=False):
    def body(a_r, g_r, b_r, w1_r, b1_r, w2_r, b2_r, w3_r, b3_r, o_r):
        agg = a_r[0] + a_r[1]
        g = g_r[...]
        b = b_r[...]
        h = _layer_norm_in(agg, g, b)
        f = _softplus(_dot_t(h, w1_r[...]) + b1_r[...])
        f = _softplus(_dot_t(f, w2_r[...]) + b2_r[...])
        f = _softplus(_dot_t(f, w3_r[...]) + b3_r[...])
        o_r[...] = _layer_norm_in(f, g, b)

    return pl.pallas_call(
        body,
        out_shape=jax.ShapeDtypeStruct((N_NODES, H), jnp.float32),
        interpret=interpret,
    )(aggp, ln_g, ln_b, W1, b1, W2, b2, W3, b3)


def kernel(atom_embs, edge_indices, pos, edge_weight, Wq, Wk, Wv, Wi, bi, Wj,
           bj, We, be, Wr, br, Wo, bo, ln_g, ln_b, W1, b1, W2, b2, W3, b3):
    src = edge_indices[0]
    dst = edge_indices[1]
    pos_pad = jnp.pad(pos, ((0, 0), (0, H - 3)))
    ew = edge_weight.reshape(N_EDGES, 1)
    src2 = src.reshape(N_EDGES, 1)
    centers = jnp.asarray(
        np.linspace(1.0, np.exp(-CUTOFF), RBF_K), dtype=jnp.float32
    ).reshape(1, RBF_K)
    r1 = lambda v: v.reshape(1, -1)

    gd, gs, pd, ps = _sc_gather(atom_embs, pos_pad, src, dst)
    Q, K, inner = _tc_prep(gd, gs, pd, ps, ew, Wq, Wk, Wv, Wi, r1(bi), Wj,
                           r1(bj), We, r1(be), Wr, r1(br), centers)
    msg = _tc_attn(Q, K, inner, src2, Wo, r1(bo))
    aggp = _sc_scatter(msg, dst, jnp.zeros((N_NODES, H), jnp.float32))
    return _tc_final(aggp, r1(ln_g), r1(ln_b), W1, r1(b1), W2, r1(b2), W3,
                     r1(b3))


# speedup vs baseline: 2.5166x; 1.0000x over previous
import functools

import jax
import jax.numpy as jnp
import numpy as np
from jax import lax
from jax.experimental import pallas as pl
from jax.experimental.pallas import tpu as pltpu
from jax.experimental.pallas import tpu_sc as plsc

H = 128
NHEAD = 8
HH = H * NHEAD
RBF_K = 64
CUTOFF = 10.0
N_NODES = 1024
N_EDGES = 2048

_NC, _NS = 2, 16
_NW = _NC * _NS
_EPW = N_EDGES // _NW


def _sc_gather(atom_embs, pos_pad, src, dst):
    mesh = plsc.VectorSubcoreMesh(core_axis_name="c", subcore_axis_name="s")

    @functools.partial(
        pl.kernel,
        out_type=(
            jax.ShapeDtypeStruct((N_EDGES, H), jnp.float32),
            jax.ShapeDtypeStruct((N_EDGES, H), jnp.float32),
            jax.ShapeDtypeStruct((N_EDGES, H), jnp.float32),
            jax.ShapeDtypeStruct((N_EDGES, H), jnp.float32),
        ),
        mesh=mesh,
        scratch_types=[
            pltpu.VMEM((_EPW,), jnp.int32),
            pltpu.VMEM((_EPW,), jnp.int32),
            pltpu.VMEM((_EPW, H), jnp.float32),
            pltpu.SemaphoreType.DMA,
        ],
    )
    def k(embs_hbm, pos_hbm, src_hbm, dst_hbm, gd_hbm, gs_hbm, pd_hbm, ps_hbm,
          idx_d, idx_s, rows, sem):
        wid = lax.axis_index("s") * _NC + lax.axis_index("c")
        base = wid * _EPW
        pltpu.sync_copy(dst_hbm.at[pl.ds(base, _EPW)], idx_d)
        pltpu.sync_copy(src_hbm.at[pl.ds(base, _EPW)], idx_s)
        pltpu.async_copy(embs_hbm.at[idx_d], rows, sem).wait()
        pltpu.sync_copy(rows, gd_hbm.at[pl.ds(base, _EPW)])
        pltpu.async_copy(embs_hbm.at[idx_s], rows, sem).wait()
        pltpu.sync_copy(rows, gs_hbm.at[pl.ds(base, _EPW)])
        pltpu.async_copy(pos_hbm.at[idx_d], rows, sem).wait()
        pltpu.sync_copy(rows, pd_hbm.at[pl.ds(base, _EPW)])
        pltpu.async_copy(pos_hbm.at[idx_s], rows, sem).wait()
        pltpu.sync_copy(rows, ps_hbm.at[pl.ds(base, _EPW)])

    return k(atom_embs, pos_pad, src, dst)


def _sc_scatter(msg, dst, zeros):
    mesh = plsc.VectorSubcoreMesh(core_axis_name="c", subcore_axis_name="s")
    rpw = N_NODES // _NS

    @functools.partial(
        pl.kernel,
        out_type=jax.ShapeDtypeStruct((_NC, N_NODES, H), jnp.float32),
        mesh=mesh,
        scratch_types=[
            pltpu.VMEM((_EPW,), jnp.int32),
            pltpu.VMEM((_EPW, H), jnp.float32),
            pltpu.VMEM_SHARED((N_NODES, H), jnp.float32),
            pltpu.SemaphoreType.DMA,
        ],
    )
    def k(msg_hbm, dst_hbm, zeros_hbm, out_hbm, idx_v, rows_v, agg_s, sem):
        cid = lax.axis_index("c")
        sid = lax.axis_index("s")
        wid = sid * _NC + cid
        base = wid * _EPW

        @pl.when(sid == 0)
        def _():
            pltpu.sync_copy(zeros_hbm, agg_s)

        plsc.subcore_barrier()
        pltpu.sync_copy(msg_hbm.at[pl.ds(base, _EPW)], rows_v)
        pltpu.sync_copy(dst_hbm.at[pl.ds(base, _EPW)], idx_v)
        pltpu.sync_copy(rows_v, agg_s.at[idx_v], add=True)
        plsc.subcore_barrier()
        pltpu.sync_copy(agg_s.at[pl.ds(sid * rpw, rpw)],
                        out_hbm.at[cid, pl.ds(sid * rpw, rpw)])

    return k(msg, dst, zeros)


def _dot_t(a, b):
    return lax.dot_general(a, b, (((1,), (1,)), ((), ())),
                           preferred_element_type=jnp.float32)


def _dot(a, b):
    return lax.dot_general(a, b, (((1,), (0,)), ((), ())),
                           preferred_element_type=jnp.float32)


_RBF_WIDTH = float((0.5 / ((1.0 - np.exp(-CUTOFF)) / RBF_K)) ** 2)


def _tc_prep(gd, gs, pd, ps, ew, Wq, Wk, Wv, Wi, bi, Wj, bj, We, be, Wr, br,
             centers, interpret=False):
    blk = 512
    grid = N_EDGES // blk

    def body(gd_r, gs_r, pd_r, ps_r, ew_r, Wq_r, Wk_r, Wv_r, Wi_r, bi_r,
             Wj_r, bj_r, We_r, be_r, Wr_r, br_r, c_r, q_o, k_o, inner_o):
        ew_b = ew_r[...]
        x_i = gd_r[...] + ew_b
        x_j = gs_r[...] + ew_b
        q_o[...] = _dot_t(x_i, Wq_r[...])
        k_o[...] = _dot_t(x_i, Wk_r[...])
        v = _dot_t(x_i, Wv_r[...])
        hi = _dot_t(x_i, Wi_r[...]) + bi_r[...]
        hj = _dot_t(x_j, Wj_r[...]) + bj_r[...]
        edge = jnp.concatenate([hi + hj, hi - hj, hi * hj], axis=1)
        diff = pd_r[...] - ps_r[...]
        dist = jnp.sqrt(jnp.sum(diff * diff, axis=1, keepdims=True))
        x = dist / CUTOFF
        x3 = x ** 3
        x4 = x3 * x
        x5 = x4 * x
        cut = jnp.where(x < 1.0, 1 - 6 * x5 + 15 * x4 - 10 * x3,
                        jnp.zeros_like(x))
        rbf = cut * jnp.exp(-_RBF_WIDTH * (jnp.exp(-dist) - c_r[...]) ** 2)
        inner_o[...] = (_dot_t(edge, We_r[...]) + be_r[...] +
                        _dot_t(rbf, Wr_r[...]) + br_r[...] + v)

    full = lambda shape: pl.BlockSpec(shape, lambda i: (0, 0))
    rows = lambda w: pl.BlockSpec((blk, w), lambda i: (i, 0))
    return pl.pallas_call(
        body,
        grid=(grid,),
        in_specs=[
            rows(H), rows(H), rows(H), rows(H), rows(1),
            full((HH, H)), full((HH, H)), full((HH, H)),
            full((H, H)), full((1, H)), full((H, H)), full((1, H)),
            full((HH, 3 * H)), full((1, HH)), full((HH, RBF_K)),
            full((1, HH)), full((1, RBF_K)),
        ],
        out_specs=[rows(HH), rows(HH), rows(HH)],
        out_shape=[
            jax.ShapeDtypeStruct((N_EDGES, HH), jnp.float32),
            jax.ShapeDtypeStruct((N_EDGES, HH), jnp.float32),
            jax.ShapeDtypeStruct((N_EDGES, HH), jnp.float32),
        ],
        interpret=interpret,
    )(gd, gs, pd, ps, ew, Wq, Wk, Wv, Wi, bi, Wj, bj, We, be, Wr, br, centers)


def _tc_attn(Q, K, inner, src2, Wo, bo, interpret=False):
    blk = 256
    grid = N_EDGES // blk
    scale = float(H) ** -0.5

    def body(q_r, k_r, inner_r, src_r, wo_r, bo_r, msg_o, p_scr):
        @pl.when(pl.program_id(0) == 0)
        def _():
            ids = lax.broadcasted_iota(jnp.int32, (N_EDGES, N_NODES), 1)
            p_scr[...] = (src_r[...] == ids).astype(jnp.bfloat16)

        logits = _dot_t(q_r[...], k_r[...]) * scale
        c = jnp.max(logits, axis=1, keepdims=True)
        e = jnp.exp(logits - c)
        p = p_scr[...]
        s = _dot(e.astype(jnp.bfloat16), p)
        denom = _dot_t(s.astype(jnp.bfloat16), p)
        out = _dot(e / denom, inner_r[...])
        msg_o[...] = _dot_t(out, wo_r[...]) + bo_r[...]

    return pl.pallas_call(
        body,
        grid=(grid,),
        in_specs=[
            pl.BlockSpec((blk, HH), lambda i: (i, 0)),
            pl.BlockSpec((N_EDGES, HH), lambda i: (0, 0)),
            pl.BlockSpec((N_EDGES, HH), lambda i: (0, 0)),
            pl.BlockSpec((N_EDGES, 1), lambda i: (0, 0)),
            pl.BlockSpec((H, HH), lambda i: (0, 0)),
            pl.BlockSpec((1, H), lambda i: (0, 0)),
        ],
        out_specs=pl.BlockSpec((blk, H), lambda i: (i, 0)),
        out_shape=jax.ShapeDtypeStruct((N_EDGES, H), jnp.float32),
        scratch_shapes=[pltpu.VMEM((N_EDGES, N_NODES), jnp.bfloat16)],
        interpret=interpret,
    )(Q, K, inner, src2, Wo, bo)


def _layer_norm_in(x, g, b, eps=1e-5):
    mu = jnp.mean(x, axis=-1, keepdims=True)
    var = jnp.mean((x - mu) ** 2, axis=-1, keepdims=True)
    return (x - mu) / jnp.sqrt(var + eps) * g + b


def _softplus(x):
    return jnp.maximum(x, 0.0) + jnp.log(1.0 + jnp.exp(-jnp.abs(x)))


def _tc_final(aggp, ln_g, ln_b, W1, b1, W2, b2, W3, b3, interpret=False):
    def body(a_r, g_r, b_r, w1_r, b1_r, w2_r, b2_r, w3_r, b3_r, o_r):
        agg = a_r[0] + a_r[1]
        g = g_r[...]
        b = b_r[...]
        h = _layer_norm_in(agg, g, b)
        f = _softplus(_dot_t(h, w1_r[...]) + b1_r[...])
        f = _softplus(_dot_t(f, w2_r[...]) + b2_r[...])
        f = _softplus(_dot_t(f, w3_r[...]) + b3_r[...])
        o_r[...] = _layer_norm_in(f, g, b)

    return pl.pallas_call(
        body,
        out_shape=jax.ShapeDtypeStruct((N_NODES, H), jnp.float32),
        interpret=interpret,
    )(aggp, ln_g, ln_b, W1, b1, W2, b2, W3, b3)


def kernel(atom_embs, edge_indices, pos, edge_weight, Wq, Wk, Wv, Wi, bi, Wj,
           bj, We, be, Wr, br, Wo, bo, ln_g, ln_b, W1, b1, W2, b2, W3, b3):
    src = edge_indices[0]
    dst = edge_indices[1]
    pos_pad = jnp.pad(pos, ((0, 0), (0, H - 3)))
    ew = edge_weight.reshape(N_EDGES, 1)
    src2 = src.reshape(N_EDGES, 1)
    centers = jnp.asarray(
        np.linspace(1.0, np.exp(-CUTOFF), RBF_K), dtype=jnp.float32
    ).reshape(1, RBF_K)
    r1 = lambda v: v.reshape(1, -1)

    gd, gs, pd, ps = _sc_gather(atom_embs, pos_pad, src, dst)
    Q, K, inner = _tc_prep(gd, gs, pd, ps, ew, Wq, Wk, Wv, Wi, r1(bi), Wj,
                           r1(bj), We, r1(be), Wr, r1(br), centers)
    msg = _tc_attn(Q, K, inner, src2, Wo, r1(bo))
    aggp = _sc_scatter(msg, dst, jnp.zeros((N_NODES, H), jnp.float32))
    return _tc_final(aggp, r1(ln_g), r1(ln_b), W1, r1(b1), W2, r1(b2), W3,
                     r1(b3))
